# Initial kernel scaffold; baseline (speedup 1.0000x reference)
import jax
import jax.numpy as jnp
from jax.experimental import pallas as pl


def kernel(x, pos_table):
    raise NotImplementedError("write your pallas kernel here")


# speedup vs baseline: 1.4598x; 1.4598x over previous
import jax
import jax.numpy as jnp
from jax.experimental import pallas as pl

MAX_LEN_ = 4096
D_MODEL_ = 1024
BATCH_ = 4
BLOCK_ = 512


def _add_block(x_ref, t_ref, o_ref):
    o_ref[...] = x_ref[...] + t_ref[...]


def kernel(x, pos_table):
    b, L, d = x.shape
    xf = x.reshape(b * L, d)
    n_blocks = (b * L) // BLOCK_
    tbl_blocks_per_batch = L // BLOCK_
    out = pl.pallas_call(
        _add_block,
        out_shape=jax.ShapeDtypeStruct((b * L, d), x.dtype),
        grid=(n_blocks,),
        in_specs=[
            pl.BlockSpec((BLOCK_, d), lambda i: (i, 0)),
            pl.BlockSpec((BLOCK_, d), lambda i: (i % tbl_blocks_per_batch, 0)),
        ],
        out_specs=pl.BlockSpec((BLOCK_, d), lambda i: (i, 0)),
    )(xf, pos_table)
    return out.reshape(b, L, d)
